# baseline (speedup 1.0000x reference)
import functools

import jax
import jax.numpy as jnp
from jax import lax
from jax.experimental import pallas as pl
from jax.experimental.pallas import tpu as pltpu, tpu_sc as plsc

N = 10000
NPAD = 10240
E = 160000
EPAD = 163840
D = 256
H = 128
TRASH = 10016
CK = 128
EPT = EPAD // 16
NCH = EPT // CK
RPT = NPAD // 16
XPT = NPAD // 32
XCK = 80
DCK = 80
ACC_R = 10112
RPA = ACC_R // 16

_mesh = plsc.VectorSubcoreMesh(core_axis_name="c", subcore_axis_name="s")
f32 = jnp.float32


def _prep_body(xp, emb, dstp, zeros1, h0, deg, deg_sh, rows_a, rows_b, xi,
               da, db, ones_v, sem_a, sem_b, ta, tb):
    c = lax.axis_index("c")
    s = lax.axis_index("s")
    wid = s * 2 + c
    for i in range(DCK // 16):
        ones_v[pl.ds(i * 16, 16)] = jnp.ones((16,), f32)
    pltpu.sync_copy(xp.at[pl.ds(wid * XPT, XPT)], xi)
    rbufs = (rows_a, rows_b)
    rsems = (sem_a, sem_b)
    nx = XPT // XCK
    for j in range(min(2, nx)):
        pltpu.async_copy(emb.at[xi.at[pl.ds(j * XCK, XCK)]],
                         rbufs[j % 2], rsems[j % 2])
    for j in range(nx):
        off = wid * XPT + j * XCK
        pltpu.make_async_copy(emb.at[xi.at[pl.ds(0, XCK)]], rbufs[j % 2],
                              rsems[j % 2]).wait()
        pltpu.sync_copy(rbufs[j % 2], h0.at[pl.ds(off, XCK)])
        if j + 2 < nx:
            pltpu.async_copy(emb.at[xi.at[pl.ds((j + 2) * XCK, XCK)]],
                             rbufs[j % 2], rsems[j % 2])

    pltpu.sync_copy(zeros1.at[pl.ds(0, RPT)], deg_sh.at[pl.ds(s * RPT, RPT)])
    plsc.subcore_barrier()
    ehalf = EPAD // 2
    base = c * ehalf + s * (ehalf // 16)
    nd = ehalf // 16 // DCK
    dbufs = (da, db)
    dsems = (ta, tb)

    def dload(j, b):
        pltpu.async_copy(dstp.at[pl.ds(base + j * DCK, DCK)], dbufs[b],
                         dsems[b])

    def dwait(b):
        pltpu.make_async_copy(dstp.at[pl.ds(0, DCK)], dbufs[b],
                              dsems[b]).wait()

    dload(0, 0)
    dload(1, 1)

    def body(gi, carry):
        j0 = 2 * gi
        for b in range(2):
            dwait(b)
            pltpu.sync_copy(ones_v, deg_sh.at[dbufs[b]], add=True)
            jn = j0 + b + 2

            @pl.when(jn < nd)
            def _():
                dload(jn, b)
        return carry
    lax.fori_loop(0, nd // 2, body, 0)
    if nd % 2 == 1:
        dwait(0)
        pltpu.sync_copy(ones_v, deg_sh.at[dbufs[0]], add=True)
    plsc.subcore_barrier()
    pltpu.sync_copy(deg_sh.at[pl.ds(s * RPT, RPT)],
                    deg.at[pl.ds(c * NPAD + s * RPT, RPT)])


_prep = pl.kernel(
    _prep_body,
    out_type=(jax.ShapeDtypeStruct((NPAD, D), f32),
              jax.ShapeDtypeStruct((2 * NPAD,), f32)),
    mesh=_mesh,
    scratch_types=[pltpu.VMEM_SHARED((NPAD,), f32),
                   pltpu.VMEM((XCK, D), f32),
                   pltpu.VMEM((XCK, D), f32),
                   pltpu.VMEM((XPT,), jnp.int32),
                   pltpu.VMEM((DCK,), jnp.int32),
                   pltpu.VMEM((DCK,), jnp.int32),
                   pltpu.VMEM((DCK,), f32),
                   pltpu.SemaphoreType.DMA,
                   pltpu.SemaphoreType.DMA,
                   pltpu.SemaphoreType.DMA,
                   pltpu.SemaphoreType.DMA],
)


NB = 2


def _msg_body(hw_flat, src4, dst3, zeros_h, acc2, acc_sh, r0, r1,
              sidx, d0, d1, g0, g1, s0, s1, q0, q1):
    c = lax.axis_index("c")
    s = lax.axis_index("s")
    w = c * 16 + s
    pltpu.sync_copy(src4.at[pl.ds(w * EPT, EPT)], sidx)
    pltpu.sync_copy(zeros_h, acc_sh.at[pl.ds(s * RPA, RPA)])
    plsc.subcore_barrier()

    bufs = (r0, r1)
    dbufs = (d0, d1)
    gsems = (g0, g1)
    ssems = (s0, s1)
    qsems = (q0, q1)

    def dload(j, b):
        pltpu.async_copy(dst3.at[pl.ds(s * NCH + j, 1)], dbufs[b],
                         qsems[b])

    def dload_wait(b):
        pltpu.make_async_copy(dst3.at[pl.ds(0, 1)], dbufs[b],
                              qsems[b]).wait()

    def gather(j, b):
        pltpu.async_copy(hw_flat.at[sidx.at[pl.ds(j * CK, CK)]], bufs[b],
                         gsems[b])

    def gather_wait(b):
        pltpu.make_async_copy(hw_flat.at[sidx.at[pl.ds(0, CK)]], bufs[b],
                              gsems[b]).wait()

    def scat(b):
        pltpu.async_copy(bufs[b], acc_sh.at[dbufs[b].at[0]], ssems[b],
                         add=True)

    def scat_wait(b):
        pltpu.make_async_copy(bufs[b], acc_sh.at[dbufs[b].at[0]],
                              ssems[b]).wait()

    for b in range(NB):
        dload(b, b)
        gather(b, b)

    def body(gi, carry):
        j0 = NB * gi
        for b in range(NB):
            gather_wait(b)
            dload_wait(b)
            scat(b)
        for b in range(NB):
            jn = j0 + b + NB
            scat_wait(b)

            @pl.when(jn < NCH)
            def _():
                dload(jn, b)
                gather(jn, b)
        return carry
    lax.fori_loop(0, NCH // NB, body, 0)
    plsc.subcore_barrier()
    pltpu.sync_copy(acc_sh.at[pl.ds(s * RPA, RPA)],
                    acc2.at[pl.ds(c * NPAD + s * RPA, RPA)])


_msg = pl.kernel(
    _msg_body,
    out_type=jax.ShapeDtypeStruct((2 * NPAD, H), f32),
    mesh=_mesh,
    scratch_types=[pltpu.VMEM_SHARED((ACC_R, H), f32)]
                  + [pltpu.VMEM((CK, H), f32)] * NB
                  + [pltpu.VMEM((EPT,), jnp.int32)]
                  + [pltpu.VMEM((1, 128), jnp.int32)] * NB
                  + [pltpu.SemaphoreType.DMA] * (3 * NB),
)


def _msg2_body(hwm_flat, hwl_flat, src4, dst3, zeros_h, accm2, accl2,
               acc_sh, r0, r1, sidx, d0, d1, g0, g1, s0, s1, q0, q1):
    c = lax.axis_index("c")
    s = lax.axis_index("s")
    w = c * 16 + s
    pltpu.sync_copy(src4.at[pl.ds(w * EPT, EPT)], sidx)
    dbufs = (d0, d1)
    qsems = (q0, q1)

    bufs = (r0, r1)
    gsems = (g0, g1)
    ssems = (s0, s1)

    for hw_flat, acc2 in ((hwm_flat, accm2), (hwl_flat, accl2)):
        pltpu.sync_copy(zeros_h, acc_sh.at[pl.ds(s * RPA, RPA)])
        plsc.subcore_barrier()

        def dload(j, b):
            pltpu.async_copy(dst3.at[pl.ds(s * NCH + j, 1)], dbufs[b],
                             qsems[b])

        def dload_wait(b):
            pltpu.make_async_copy(dst3.at[pl.ds(0, 1)], dbufs[b],
                                  qsems[b]).wait()

        def gather(j, b):
            pltpu.async_copy(hw_flat.at[sidx.at[pl.ds(j * CK, CK)]],
                             bufs[b], gsems[b])

        def gather_wait(b):
            pltpu.make_async_copy(hw_flat.at[sidx.at[pl.ds(0, CK)]],
                                  bufs[b], gsems[b]).wait()

        def scat(b):
            pltpu.async_copy(bufs[b], acc_sh.at[dbufs[b].at[0]], ssems[b],
                             add=True)

        def scat_wait(b):
            pltpu.make_async_copy(bufs[b], acc_sh.at[dbufs[b].at[0]],
                                  ssems[b]).wait()

        for b in range(NB):
            dload(b, b)
            gather(b, b)

        def body(gi, carry):
            j0 = NB * gi
            for b in range(NB):
                gather_wait(b)
                dload_wait(b)
                scat(b)
            for b in range(NB):
                jn = j0 + b + NB
                scat_wait(b)

                @pl.when(jn < NCH)
                def _():
                    dload(jn, b)
                    gather(jn, b)
            return carry
        lax.fori_loop(0, NCH // NB, body, 0)
        plsc.subcore_barrier()
        pltpu.sync_copy(acc_sh.at[pl.ds(s * RPA, RPA)],
                        acc2.at[pl.ds(c * NPAD + s * RPA, RPA)])
        plsc.subcore_barrier()


_msg2 = pl.kernel(
    _msg2_body,
    out_type=(jax.ShapeDtypeStruct((2 * NPAD, H), f32),) * 2,
    mesh=_mesh,
    scratch_types=[pltpu.VMEM_SHARED((ACC_R, H), f32)]
                  + [pltpu.VMEM((CK, H), f32)] * NB
                  + [pltpu.VMEM((EPT,), jnp.int32)]
                  + [pltpu.VMEM((1, 128), jnp.int32)] * NB
                  + [pltpu.SemaphoreType.DMA] * (3 * NB),
)


_GRID = NPAD // 256


def _k1_first_body(h_ref, w_ref, deg_ref, hw_ref, dv_ref):
    dinv = lax.rsqrt(deg_ref[...] + 1.0)
    hw = jnp.dot(h_ref[...].astype(jnp.bfloat16), w_ref[...].astype(jnp.bfloat16),
                 preferred_element_type=f32)
    dinvb = jnp.concatenate([dinv, dinv], axis=1)
    hw_ref[...] = (hw * dinvb).reshape(256, 2, H)
    dv_ref[...] = dinv


def _k1_first(h0, W0, degb):
    return pl.pallas_call(
        _k1_first_body,
        grid=(_GRID,),
        in_specs=[pl.BlockSpec((256, D), lambda i: (i, 0)),
                  pl.BlockSpec((D, D), lambda i: (0, 0)),
                  pl.BlockSpec((256, H), lambda i: (i, 0))],
        out_specs=[pl.BlockSpec((256, 2, H), lambda i: (i, 0, 0)),
                   pl.BlockSpec((256, H), lambda i: (i, 0))],
        out_shape=[jax.ShapeDtypeStruct((NPAD, 2, H), f32),
                   jax.ShapeDtypeStruct((NPAD, H), f32)],
    )(h0, W0, degb)


def _k2_body(aL, aR, h3, dv, b, z_ref, st_ref):
    pid = pl.program_id(0)
    dinv = dv[...]
    hw = h3[...]
    zL = dinv * (aL[...] + hw[:, 0, :])
    zR = dinv * (aR[...] + hw[:, 1, :])
    z = jnp.concatenate([zL, zR], axis=1) + b[...]
    z_ref[...] = z
    rows = pid * 256 + lax.broadcasted_iota(jnp.int32, (256, 1), 0)
    zm = jnp.where(rows < N, z, 0.0)
    s1 = jnp.sum(zm, axis=0, keepdims=True)
    s2 = jnp.sum(zm * zm, axis=0, keepdims=True)
    part = jnp.concatenate([jnp.broadcast_to(s1, (4, D)),
                            jnp.broadcast_to(s2, (4, D))], axis=0)

    @pl.when(pid == 0)
    def _():
        st_ref[...] = jnp.zeros((8, D), f32)
    st_ref[...] += part


def _k2(acc2, hw3, dv, b):
    return pl.pallas_call(
        _k2_body,
        grid=(_GRID,),
        in_specs=[pl.BlockSpec((256, H), lambda i: (i, 0)),
                  pl.BlockSpec((256, H), lambda i: (i + _GRID, 0)),
                  pl.BlockSpec((256, 2, H), lambda i: (i, 0, 0)),
                  pl.BlockSpec((256, H), lambda i: (i, 0)),
                  pl.BlockSpec((1, D), lambda i: (0, 0))],
        out_specs=[pl.BlockSpec((256, D), lambda i: (i, 0)),
                   pl.BlockSpec((8, D), lambda i: (0, 0))],
        out_shape=[jax.ShapeDtypeStruct((NPAD, D), f32),
                   jax.ShapeDtypeStruct((8, D), f32)],
        compiler_params=pltpu.CompilerParams(
            dimension_semantics=("arbitrary",)),
    )(acc2, acc2, hw3, dv, b)


def _bn_relu(z_ref, st_ref, g_ref, be_ref):
    st = st_ref[...]
    mu = st[0:1, :] * (1.0 / N)
    ex2 = st[4:5, :] * (1.0 / N)
    var = ex2 - mu * mu
    scale = g_ref[...] * lax.rsqrt(var + 1e-5)
    return jnp.maximum(scale * (z_ref[...] - mu) + be_ref[...], 0.0)


def _k1_body(z_ref, st_ref, g_ref, be_ref, w_ref, dv_ref, hw_ref):
    h = _bn_relu(z_ref, st_ref, g_ref, be_ref)
    hw = jnp.dot(h, w_ref[...], preferred_element_type=f32)
    dinv = dv_ref[...]
    dinvb = jnp.concatenate([dinv, dinv], axis=1)
    hw_ref[...] = (hw * dinvb).reshape(256, 2, H)


def _k1(z, st, g, be, W, dv):
    return pl.pallas_call(
        _k1_body,
        grid=(_GRID,),
        in_specs=[pl.BlockSpec((256, D), lambda i: (i, 0)),
                  pl.BlockSpec((8, D), lambda i: (0, 0)),
                  pl.BlockSpec((1, D), lambda i: (0, 0)),
                  pl.BlockSpec((1, D), lambda i: (0, 0)),
                  pl.BlockSpec((D, D), lambda i: (0, 0)),
                  pl.BlockSpec((256, H), lambda i: (i, 0))],
        out_specs=[pl.BlockSpec((256, 2, H), lambda i: (i, 0, 0))],
        out_shape=[jax.ShapeDtypeStruct((NPAD, 2, H), f32)],
    )(z, st, g, be, W, dv)[0]


def _k1_final_body(z_ref, st_ref, g_ref, be_ref, w_ref, dv_ref,
                   hm_ref, hl_ref):
    h = _bn_relu(z_ref, st_ref, g_ref, be_ref)
    hw = jnp.dot(h, w_ref[...], preferred_element_type=f32)
    dinv = dv_ref[...]
    dinvb = jnp.concatenate([dinv, dinv], axis=1)
    hm_ref[...] = (hw[:, :D] * dinvb).reshape(256, 2, H)
    hl_ref[...] = (hw[:, D:] * dinvb).reshape(256, 2, H)


def _k1_final(z, st, g, be, Wml, dv):
    return pl.pallas_call(
        _k1_final_body,
        grid=(_GRID,),
        in_specs=[pl.BlockSpec((256, D), lambda i: (i, 0)),
                  pl.BlockSpec((8, D), lambda i: (0, 0)),
                  pl.BlockSpec((1, D), lambda i: (0, 0)),
                  pl.BlockSpec((1, D), lambda i: (0, 0)),
                  pl.BlockSpec((D, 2 * D), lambda i: (0, 0)),
                  pl.BlockSpec((256, H), lambda i: (i, 0))],
        out_specs=[pl.BlockSpec((256, 2, H), lambda i: (i, 0, 0))] * 2,
        out_shape=[jax.ShapeDtypeStruct((NPAD, 2, H), f32)] * 2,
    )(z, st, g, be, Wml, dv)


def _fused_body(aL, aR, h3, dv, b, g, be, w_ref, *rest):
    outs = rest[:-2]
    z_sc, st_sc = rest[-2:]
    t = pl.program_id(0)
    i = pl.program_id(1)

    @pl.when(t == 0)
    def _():
        dinv = dv[...]
        hw = h3[...]
        zL = dinv * (aL[...] + hw[:, 0, :])
        zR = dinv * (aR[...] + hw[:, 1, :])
        z = jnp.concatenate([zL, zR], axis=1) + b[...]
        z_sc[pl.ds(i * 256, 256), :] = z
        rows = i * 256 + lax.broadcasted_iota(jnp.int32, (256, 1), 0)
        zm = jnp.where(rows < N, z, 0.0)
        s1 = jnp.sum(zm, axis=0, keepdims=True)
        s2 = jnp.sum(zm * zm, axis=0, keepdims=True)
        part = jnp.concatenate([jnp.broadcast_to(s1, (4, D)),
                                jnp.broadcast_to(s2, (4, D))], axis=0)

        @pl.when(i == 0)
        def _():
            st_sc[...] = jnp.zeros((8, D), f32)
        st_sc[...] += part

    @pl.when(t == 1)
    def _():
        st = st_sc[...]
        mu = st[0:1, :] * (1.0 / N)
        ex2 = st[4:5, :] * (1.0 / N)
        var = ex2 - mu * mu
        scale = g[...] * lax.rsqrt(var + 1e-5)
        z = z_sc[pl.ds(i * 256, 256), :]
        h = jnp.maximum(scale * (z - mu) + be[...], 0.0)
        hw = jnp.dot(h.astype(jnp.bfloat16), w_ref[...].astype(jnp.bfloat16),
                     preferred_element_type=f32)
        dinv = dv[...]
        dinvb = jnp.concatenate([dinv, dinv], axis=1)
        nw = w_ref.shape[1] // D
        for k in range(nw):
            outs[k][...] = (hw[:, k * D:(k + 1) * D] * dinvb).reshape(256, 2, H)


def _fused(acc2, hw3, dv, b, g, be, W):
    nw = W.shape[1] // D
    bi = lambda t, i: ((1 - t) * i, 0)
    bi3 = lambda t, i: ((1 - t) * i, 0, 0)
    ba = lambda t, i: (i, 0)
    bc = lambda t, i: (0, 0)
    return pl.pallas_call(
        _fused_body,
        grid=(2, _GRID),
        in_specs=[pl.BlockSpec((256, H), bi),
                  pl.BlockSpec((256, H), lambda t, i: ((1 - t) * i + _GRID, 0)),
                  pl.BlockSpec((256, 2, H), bi3),
                  pl.BlockSpec((256, H), ba),
                  pl.BlockSpec((1, D), bc),
                  pl.BlockSpec((1, D), bc),
                  pl.BlockSpec((1, D), bc),
                  pl.BlockSpec((D, nw * D), bc)],
        out_specs=[pl.BlockSpec((256, 2, H), lambda t, i: (i, 0, 0))] * nw,
        out_shape=[jax.ShapeDtypeStruct((NPAD, 2, H), f32)] * nw,
        scratch_shapes=[pltpu.VMEM((NPAD, D), f32),
                        pltpu.VMEM((8, D), f32)],
        compiler_params=pltpu.CompilerParams(
            dimension_semantics=("arbitrary", "arbitrary")),
    )(acc2, acc2, hw3, dv, b, g, be, W)


def _k2_final_body(amL, amR, alL, alR, hm3, hl3, dv, bm, bl, mu_ref, ls_ref):
    dinv = dv[...]
    hm = hm3[...]
    hl = hl3[...]
    muL = dinv * (amL[...] + hm[:, 0, :])
    muR = dinv * (amR[...] + hm[:, 1, :])
    lsL = dinv * (alL[...] + hl[:, 0, :])
    lsR = dinv * (alR[...] + hl[:, 1, :])
    mu_ref[...] = jnp.concatenate([muL, muR], axis=1) + bm[...]
    ls_ref[...] = jnp.concatenate([lsL, lsR], axis=1) + bl[...]


def _k2_final(accm2, accl2, hm3, hl3, dv, bm, bl):
    bsl = pl.BlockSpec((256, H), lambda i: (i, 0))
    bsr = pl.BlockSpec((256, H), lambda i: (i + _GRID, 0))
    bs3 = pl.BlockSpec((256, 2, H), lambda i: (i, 0, 0))
    bb = pl.BlockSpec((1, D), lambda i: (0, 0))
    return pl.pallas_call(
        _k2_final_body,
        grid=(_GRID,),
        in_specs=[bsl, bsr, bsl, bsr, bs3, bs3, bsl, bb, bb],
        out_specs=[pl.BlockSpec((256, D), lambda i: (i, 0))] * 2,
        out_shape=[jax.ShapeDtypeStruct((NPAD, D), f32)] * 2,
    )(accm2, accm2, accl2, accl2, hm3, hl3, dv, bm, bl)


def kernel(x, edge_index, emb, convW, convB, bnG, bnB, Wmu, bmu, Wls, bls):
    src, dst = edge_index[0], edge_index[1]
    srcp = jnp.concatenate([src, jnp.zeros((EPAD - E,), jnp.int32)])
    dstp = jnp.concatenate([dst, jnp.full((EPAD - E,), TRASH, jnp.int32)])
    src4 = jnp.concatenate([2 * srcp, 2 * srcp + 1])
    dst3 = dstp.reshape(16 * NCH, 128)
    xp = jnp.concatenate([x, jnp.zeros((NPAD - N,), jnp.int32)])
    zeros1 = jnp.zeros((RPT,), f32)
    zeros_h = jnp.zeros((RPA, H), f32)

    h0, deg2 = _prep(xp, emb, dstp, zeros1)
    degb = jnp.broadcast_to((deg2[:NPAD] + deg2[NPAD:])[:, None], (NPAD, H))

    hw3, dv = _k1_first(h0, convW[0], degb)
    Wml = jnp.concatenate([Wmu, Wls], axis=1)
    for i in range(4):
        acc2 = _msg(hw3.reshape(2 * NPAD, H), src4, dst3, zeros_h)
        W_next = convW[i + 1] if i < 3 else Wml
        outs = _fused(acc2, hw3, dv, convB[i][None, :], bnG[i][None, :],
                      bnB[i][None, :], W_next)
        if i < 3:
            hw3 = outs[0]
        else:
            hm3, hl3 = outs
    accm2, accl2 = _msg2(hm3.reshape(2 * NPAD, H), hl3.reshape(2 * NPAD, H),
                         src4, dst3, zeros_h)
    mu_out, ls_out = _k2_final(accm2, accl2, hm3, hl3, dv,
                               bmu[None, :], bls[None, :])
    return mu_out[:N], ls_out[:N]


# speedup vs baseline: 1.4524x; 1.4524x over previous
import functools

import jax
import jax.numpy as jnp
from jax import lax
from jax.experimental import pallas as pl
from jax.experimental.pallas import tpu as pltpu, tpu_sc as plsc

N = 10000
NPAD = 10240
E = 160000
EPAD = 161280
D = 256
H = 128
TRASH = 10016
CK = 112
EPT = EPAD // 16
NCH = EPT // CK
RPT = NPAD // 16
XPT = NPAD // 32
XCK = 80
DCK = 80
ACC_R = 10112
RPA = ACC_R // 16

_mesh = plsc.VectorSubcoreMesh(core_axis_name="c", subcore_axis_name="s")
f32 = jnp.float32


def _prep_body(xp, emb, dstp, zeros1, h0, deg, deg_sh, rows_a, rows_b, xi,
               da, db, ones_v, sem_a, sem_b, ta, tb):
    c = lax.axis_index("c")
    s = lax.axis_index("s")
    wid = s * 2 + c
    for i in range(DCK // 16):
        ones_v[pl.ds(i * 16, 16)] = jnp.ones((16,), f32)
    pltpu.sync_copy(xp.at[pl.ds(wid * XPT, XPT)], xi)
    rbufs = (rows_a, rows_b)
    rsems = (sem_a, sem_b)
    nx = XPT // XCK
    for j in range(min(2, nx)):
        pltpu.async_copy(emb.at[xi.at[pl.ds(j * XCK, XCK)]],
                         rbufs[j % 2], rsems[j % 2])
    for j in range(nx):
        off = wid * XPT + j * XCK
        pltpu.make_async_copy(emb.at[xi.at[pl.ds(0, XCK)]], rbufs[j % 2],
                              rsems[j % 2]).wait()
        pltpu.sync_copy(rbufs[j % 2], h0.at[pl.ds(off, XCK)])
        if j + 2 < nx:
            pltpu.async_copy(emb.at[xi.at[pl.ds((j + 2) * XCK, XCK)]],
                             rbufs[j % 2], rsems[j % 2])

    pltpu.sync_copy(zeros1.at[pl.ds(0, RPT)], deg_sh.at[pl.ds(s * RPT, RPT)])
    plsc.subcore_barrier()
    ehalf = EPAD // 2
    base = c * ehalf + s * (ehalf // 16)
    nd = ehalf // 16 // DCK
    dbufs = (da, db)
    dsems = (ta, tb)

    def dload(j, b):
        pltpu.async_copy(dstp.at[pl.ds(base + j * DCK, DCK)], dbufs[b],
                         dsems[b])

    def dwait(b):
        pltpu.make_async_copy(dstp.at[pl.ds(0, DCK)], dbufs[b],
                              dsems[b]).wait()

    dload(0, 0)
    dload(1, 1)

    def body(gi, carry):
        j0 = 2 * gi
        for b in range(2):
            dwait(b)
            pltpu.sync_copy(ones_v, deg_sh.at[dbufs[b]], add=True)
            jn = j0 + b + 2

            @pl.when(jn < nd)
            def _():
                dload(jn, b)
        return carry
    lax.fori_loop(0, nd // 2, body, 0)
    if nd % 2 == 1:
        dwait(0)
        pltpu.sync_copy(ones_v, deg_sh.at[dbufs[0]], add=True)
    plsc.subcore_barrier()
    pltpu.sync_copy(deg_sh.at[pl.ds(s * RPT, RPT)],
                    deg.at[pl.ds(c * NPAD + s * RPT, RPT)])


_prep = pl.kernel(
    _prep_body,
    out_type=(jax.ShapeDtypeStruct((NPAD, D), f32),
              jax.ShapeDtypeStruct((2 * NPAD,), f32)),
    mesh=_mesh,
    scratch_types=[pltpu.VMEM_SHARED((NPAD,), f32),
                   pltpu.VMEM((XCK, D), f32),
                   pltpu.VMEM((XCK, D), f32),
                   pltpu.VMEM((XPT,), jnp.int32),
                   pltpu.VMEM((DCK,), jnp.int32),
                   pltpu.VMEM((DCK,), jnp.int32),
                   pltpu.VMEM((DCK,), f32),
                   pltpu.SemaphoreType.DMA,
                   pltpu.SemaphoreType.DMA,
                   pltpu.SemaphoreType.DMA,
                   pltpu.SemaphoreType.DMA],
)


NB = 2


def _msg_body(hw_flat, src4, dst4, zeros_h, acc2, acc_sh, r0, r1,
              sidx, didx, g0, g1, s0, s1):
    c = lax.axis_index("c")
    s = lax.axis_index("s")
    w = c * 16 + s
    pltpu.sync_copy(src4.at[pl.ds(w * EPT, EPT)], sidx)
    pltpu.sync_copy(dst4.at[pl.ds(s * EPT, EPT)], didx)
    pltpu.sync_copy(zeros_h, acc_sh.at[pl.ds(s * RPA, RPA)])
    plsc.subcore_barrier()

    bufs = (r0, r1)
    gsems = (g0, g1)
    ssems = (s0, s1)

    def gather(j, b):
        pltpu.async_copy(hw_flat.at[sidx.at[pl.ds(j * CK, CK)]], bufs[b],
                         gsems[b])

    def gather_wait(b):
        pltpu.make_async_copy(hw_flat.at[sidx.at[pl.ds(0, CK)]], bufs[b],
                              gsems[b]).wait()

    def scat(j, b):
        pltpu.async_copy(bufs[b], acc_sh.at[didx.at[pl.ds(j * CK, CK)]],
                         ssems[b], add=True)

    def scat_wait(b):
        pltpu.make_async_copy(bufs[b], acc_sh.at[didx.at[pl.ds(0, CK)]],
                              ssems[b]).wait()

    for b in range(NB):
        gather(b, b)

    def body(gi, carry):
        j0 = NB * gi
        for b in range(NB):
            gather_wait(b)
            scat(j0 + b, b)
        for b in range(NB):
            jn = j0 + b + NB
            scat_wait(b)

            @pl.when(jn < NCH)
            def _():
                gather(jn, b)
        return carry
    lax.fori_loop(0, NCH // NB, body, 0)
    plsc.subcore_barrier()
    pltpu.sync_copy(acc_sh.at[pl.ds(s * RPA, RPA)],
                    acc2.at[pl.ds(c * NPAD + s * RPA, RPA)])


_msg = pl.kernel(
    _msg_body,
    out_type=jax.ShapeDtypeStruct((2 * NPAD, H), f32),
    mesh=_mesh,
    scratch_types=[pltpu.VMEM_SHARED((ACC_R, H), f32)]
                  + [pltpu.VMEM((CK, H), f32)] * NB
                  + [pltpu.VMEM((EPT,), jnp.int32)] * 2
                  + [pltpu.SemaphoreType.DMA] * (2 * NB),
)


def _msg2_body(hwm_flat, hwl_flat, src4, dst4, zeros_h, accm2, accl2,
               acc_sh, r0, r1, sidx, didx, g0, g1, s0, s1):
    c = lax.axis_index("c")
    s = lax.axis_index("s")
    w = c * 16 + s
    pltpu.sync_copy(src4.at[pl.ds(w * EPT, EPT)], sidx)
    pltpu.sync_copy(dst4.at[pl.ds(s * EPT, EPT)], didx)

    bufs = (r0, r1)
    gsems = (g0, g1)
    ssems = (s0, s1)

    for hw_flat, acc2 in ((hwm_flat, accm2), (hwl_flat, accl2)):
        pltpu.sync_copy(zeros_h, acc_sh.at[pl.ds(s * RPA, RPA)])
        plsc.subcore_barrier()

        def gather(j, b):
            pltpu.async_copy(hw_flat.at[sidx.at[pl.ds(j * CK, CK)]],
                             bufs[b], gsems[b])

        def gather_wait(b):
            pltpu.make_async_copy(hw_flat.at[sidx.at[pl.ds(0, CK)]],
                                  bufs[b], gsems[b]).wait()

        def scat(j, b):
            pltpu.async_copy(bufs[b], acc_sh.at[didx.at[pl.ds(j * CK, CK)]],
                             ssems[b], add=True)

        def scat_wait(b):
            pltpu.make_async_copy(bufs[b], acc_sh.at[didx.at[pl.ds(0, CK)]],
                                  ssems[b]).wait()

        for b in range(NB):
            gather(b, b)

        def body(gi, carry):
            j0 = NB * gi
            for b in range(NB):
                gather_wait(b)
                scat(j0 + b, b)
            for b in range(NB):
                jn = j0 + b + NB
                scat_wait(b)

                @pl.when(jn < NCH)
                def _():
                    gather(jn, b)
            return carry
        lax.fori_loop(0, NCH // NB, body, 0)
        plsc.subcore_barrier()
        pltpu.sync_copy(acc_sh.at[pl.ds(s * RPA, RPA)],
                        acc2.at[pl.ds(c * NPAD + s * RPA, RPA)])
        plsc.subcore_barrier()


_msg2 = pl.kernel(
    _msg2_body,
    out_type=(jax.ShapeDtypeStruct((2 * NPAD, H), f32),) * 2,
    mesh=_mesh,
    scratch_types=[pltpu.VMEM_SHARED((ACC_R, H), f32)]
                  + [pltpu.VMEM((CK, H), f32)] * NB
                  + [pltpu.VMEM((EPT,), jnp.int32)] * 2
                  + [pltpu.SemaphoreType.DMA] * (2 * NB),
)


_GRID = NPAD // 256


def _k1_first_body(h_ref, w_ref, deg_ref, hw_ref, dv_ref):
    dinv = lax.rsqrt(deg_ref[...] + 1.0)
    hw = jnp.dot(h_ref[...].astype(jnp.bfloat16), w_ref[...].astype(jnp.bfloat16),
                 preferred_element_type=f32)
    dinvb = jnp.concatenate([dinv, dinv], axis=1)
    hw_ref[...] = (hw * dinvb).reshape(256, 2, H)
    dv_ref[...] = dinv


def _k1_first(h0, W0, degb):
    return pl.pallas_call(
        _k1_first_body,
        grid=(_GRID,),
        in_specs=[pl.BlockSpec((256, D), lambda i: (i, 0)),
                  pl.BlockSpec((D, D), lambda i: (0, 0)),
                  pl.BlockSpec((256, H), lambda i: (i, 0))],
        out_specs=[pl.BlockSpec((256, 2, H), lambda i: (i, 0, 0)),
                   pl.BlockSpec((256, H), lambda i: (i, 0))],
        out_shape=[jax.ShapeDtypeStruct((NPAD, 2, H), f32),
                   jax.ShapeDtypeStruct((NPAD, H), f32)],
    )(h0, W0, degb)


def _k2_body(aL, aR, h3, dv, b, z_ref, st_ref):
    pid = pl.program_id(0)
    dinv = dv[...]
    hw = h3[...]
    zL = dinv * (aL[...] + hw[:, 0, :])
    zR = dinv * (aR[...] + hw[:, 1, :])
    z = jnp.concatenate([zL, zR], axis=1) + b[...]
    z_ref[...] = z
    rows = pid * 256 + lax.broadcasted_iota(jnp.int32, (256, 1), 0)
    zm = jnp.where(rows < N, z, 0.0)
    s1 = jnp.sum(zm, axis=0, keepdims=True)
    s2 = jnp.sum(zm * zm, axis=0, keepdims=True)
    part = jnp.concatenate([jnp.broadcast_to(s1, (4, D)),
                            jnp.broadcast_to(s2, (4, D))], axis=0)

    @pl.when(pid == 0)
    def _():
        st_ref[...] = jnp.zeros((8, D), f32)
    st_ref[...] += part


def _k2(acc2, hw3, dv, b):
    return pl.pallas_call(
        _k2_body,
        grid=(_GRID,),
        in_specs=[pl.BlockSpec((256, H), lambda i: (i, 0)),
                  pl.BlockSpec((256, H), lambda i: (i + _GRID, 0)),
                  pl.BlockSpec((256, 2, H), lambda i: (i, 0, 0)),
                  pl.BlockSpec((256, H), lambda i: (i, 0)),
                  pl.BlockSpec((1, D), lambda i: (0, 0))],
        out_specs=[pl.BlockSpec((256, D), lambda i: (i, 0)),
                   pl.BlockSpec((8, D), lambda i: (0, 0))],
        out_shape=[jax.ShapeDtypeStruct((NPAD, D), f32),
                   jax.ShapeDtypeStruct((8, D), f32)],
        compiler_params=pltpu.CompilerParams(
            dimension_semantics=("arbitrary",)),
    )(acc2, acc2, hw3, dv, b)


def _bn_relu(z_ref, st_ref, g_ref, be_ref):
    st = st_ref[...]
    mu = st[0:1, :] * (1.0 / N)
    ex2 = st[4:5, :] * (1.0 / N)
    var = ex2 - mu * mu
    scale = g_ref[...] * lax.rsqrt(var + 1e-5)
    return jnp.maximum(scale * (z_ref[...] - mu) + be_ref[...], 0.0)


def _k1_body(z_ref, st_ref, g_ref, be_ref, w_ref, dv_ref, hw_ref):
    h = _bn_relu(z_ref, st_ref, g_ref, be_ref)
    hw = jnp.dot(h, w_ref[...], preferred_element_type=f32)
    dinv = dv_ref[...]
    dinvb = jnp.concatenate([dinv, dinv], axis=1)
    hw_ref[...] = (hw * dinvb).reshape(256, 2, H)


def _k1(z, st, g, be, W, dv):
    return pl.pallas_call(
        _k1_body,
        grid=(_GRID,),
        in_specs=[pl.BlockSpec((256, D), lambda i: (i, 0)),
                  pl.BlockSpec((8, D), lambda i: (0, 0)),
                  pl.BlockSpec((1, D), lambda i: (0, 0)),
                  pl.BlockSpec((1, D), lambda i: (0, 0)),
                  pl.BlockSpec((D, D), lambda i: (0, 0)),
                  pl.BlockSpec((256, H), lambda i: (i, 0))],
        out_specs=[pl.BlockSpec((256, 2, H), lambda i: (i, 0, 0))],
        out_shape=[jax.ShapeDtypeStruct((NPAD, 2, H), f32)],
    )(z, st, g, be, W, dv)[0]


def _k1_final_body(z_ref, st_ref, g_ref, be_ref, w_ref, dv_ref,
                   hm_ref, hl_ref):
    h = _bn_relu(z_ref, st_ref, g_ref, be_ref)
    hw = jnp.dot(h, w_ref[...], preferred_element_type=f32)
    dinv = dv_ref[...]
    dinvb = jnp.concatenate([dinv, dinv], axis=1)
    hm_ref[...] = (hw[:, :D] * dinvb).reshape(256, 2, H)
    hl_ref[...] = (hw[:, D:] * dinvb).reshape(256, 2, H)


def _k1_final(z, st, g, be, Wml, dv):
    return pl.pallas_call(
        _k1_final_body,
        grid=(_GRID,),
        in_specs=[pl.BlockSpec((256, D), lambda i: (i, 0)),
                  pl.BlockSpec((8, D), lambda i: (0, 0)),
                  pl.BlockSpec((1, D), lambda i: (0, 0)),
                  pl.BlockSpec((1, D), lambda i: (0, 0)),
                  pl.BlockSpec((D, 2 * D), lambda i: (0, 0)),
                  pl.BlockSpec((256, H), lambda i: (i, 0))],
        out_specs=[pl.BlockSpec((256, 2, H), lambda i: (i, 0, 0))] * 2,
        out_shape=[jax.ShapeDtypeStruct((NPAD, 2, H), f32)] * 2,
    )(z, st, g, be, Wml, dv)


def _fused_body(aL, aR, h3, dv, b, g, be, w_ref, *rest):
    outs = rest[:-2]
    z_sc, st_sc = rest[-2:]
    t = pl.program_id(0)
    i = pl.program_id(1)

    @pl.when(t == 0)
    def _():
        dinv = dv[...]
        hw = h3[...]
        zL = dinv * (aL[...] + hw[:, 0, :])
        zR = dinv * (aR[...] + hw[:, 1, :])
        z = jnp.concatenate([zL, zR], axis=1) + b[...]
        z_sc[pl.ds(i * 256, 256), :] = z
        rows = i * 256 + lax.broadcasted_iota(jnp.int32, (256, 1), 0)
        zm = jnp.where(rows < N, z, 0.0)
        s1 = jnp.sum(zm, axis=0, keepdims=True)
        s2 = jnp.sum(zm * zm, axis=0, keepdims=True)
        part = jnp.concatenate([jnp.broadcast_to(s1, (4, D)),
                                jnp.broadcast_to(s2, (4, D))], axis=0)

        @pl.when(i == 0)
        def _():
            st_sc[...] = jnp.zeros((8, D), f32)
        st_sc[...] += part

    @pl.when(t == 1)
    def _():
        st = st_sc[...]
        mu = st[0:1, :] * (1.0 / N)
        ex2 = st[4:5, :] * (1.0 / N)
        var = ex2 - mu * mu
        scale = g[...] * lax.rsqrt(var + 1e-5)
        z = z_sc[pl.ds(i * 256, 256), :]
        h = jnp.maximum(scale * (z - mu) + be[...], 0.0)
        hw = jnp.dot(h.astype(jnp.bfloat16), w_ref[...].astype(jnp.bfloat16),
                     preferred_element_type=f32)
        dinv = dv[...]
        dinvb = jnp.concatenate([dinv, dinv], axis=1)
        nw = w_ref.shape[1] // D
        for k in range(nw):
            outs[k][...] = (hw[:, k * D:(k + 1) * D] * dinvb).reshape(256, 2, H)


def _fused(acc2, hw3, dv, b, g, be, W):
    nw = W.shape[1] // D
    bi = lambda t, i: ((1 - t) * i, 0)
    bi3 = lambda t, i: ((1 - t) * i, 0, 0)
    ba = lambda t, i: (i, 0)
    bc = lambda t, i: (0, 0)
    return pl.pallas_call(
        _fused_body,
        grid=(2, _GRID),
        in_specs=[pl.BlockSpec((256, H), bi),
                  pl.BlockSpec((256, H), lambda t, i: ((1 - t) * i + _GRID, 0)),
                  pl.BlockSpec((256, 2, H), bi3),
                  pl.BlockSpec((256, H), ba),
                  pl.BlockSpec((1, D), bc),
                  pl.BlockSpec((1, D), bc),
                  pl.BlockSpec((1, D), bc),
                  pl.BlockSpec((D, nw * D), bc)],
        out_specs=[pl.BlockSpec((256, 2, H), lambda t, i: (i, 0, 0))] * nw,
        out_shape=[jax.ShapeDtypeStruct((NPAD, 2, H), f32)] * nw,
        scratch_shapes=[pltpu.VMEM((NPAD, D), f32),
                        pltpu.VMEM((8, D), f32)],
        compiler_params=pltpu.CompilerParams(
            dimension_semantics=("arbitrary", "arbitrary")),
    )(acc2, acc2, hw3, dv, b, g, be, W)


def _k2_final_body(amL, amR, alL, alR, hm3, hl3, dv, bm, bl, mu_ref, ls_ref):
    dinv = dv[...]
    hm = hm3[...]
    hl = hl3[...]
    muL = dinv * (amL[...] + hm[:, 0, :])
    muR = dinv * (amR[...] + hm[:, 1, :])
    lsL = dinv * (alL[...] + hl[:, 0, :])
    lsR = dinv * (alR[...] + hl[:, 1, :])
    mu_ref[...] = jnp.concatenate([muL, muR], axis=1) + bm[...]
    ls_ref[...] = jnp.concatenate([lsL, lsR], axis=1) + bl[...]


def _k2_final(accm2, accl2, hm3, hl3, dv, bm, bl):
    bsl = pl.BlockSpec((256, H), lambda i: (i, 0))
    bsr = pl.BlockSpec((256, H), lambda i: (i + _GRID, 0))
    bs3 = pl.BlockSpec((256, 2, H), lambda i: (i, 0, 0))
    bb = pl.BlockSpec((1, D), lambda i: (0, 0))
    return pl.pallas_call(
        _k2_final_body,
        grid=(_GRID,),
        in_specs=[bsl, bsr, bsl, bsr, bs3, bs3, bsl, bb, bb],
        out_specs=[pl.BlockSpec((256, D), lambda i: (i, 0))] * 2,
        out_shape=[jax.ShapeDtypeStruct((NPAD, D), f32)] * 2,
    )(accm2, accm2, accl2, accl2, hm3, hl3, dv, bm, bl)


def kernel(x, edge_index, emb, convW, convB, bnG, bnB, Wmu, bmu, Wls, bls):
    src, dst = edge_index[0], edge_index[1]
    srcp = jnp.concatenate([src, jnp.zeros((EPAD - E,), jnp.int32)])
    dstp = jnp.concatenate([dst, jnp.full((EPAD - E,), TRASH, jnp.int32)])
    src4 = jnp.concatenate([2 * srcp, 2 * srcp + 1])
    xp = jnp.concatenate([x, jnp.zeros((NPAD - N,), jnp.int32)])
    zeros1 = jnp.zeros((RPT,), f32)
    zeros_h = jnp.zeros((RPA, H), f32)

    h0, deg2 = _prep(xp, emb, dstp, zeros1)
    degb = jnp.broadcast_to((deg2[:NPAD] + deg2[NPAD:])[:, None], (NPAD, H))

    hw3, dv = _k1_first(h0, convW[0], degb)
    Wml = jnp.concatenate([Wmu, Wls], axis=1)
    for i in range(4):
        acc2 = _msg(hw3.reshape(2 * NPAD, H), src4, dstp, zeros_h)
        W_next = convW[i + 1] if i < 3 else Wml
        outs = _fused(acc2, hw3, dv, convB[i][None, :], bnG[i][None, :],
                      bnB[i][None, :], W_next)
        if i < 3:
            hw3 = outs[0]
        else:
            hm3, hl3 = outs
    accm2, accl2 = _msg2(hm3.reshape(2 * NPAD, H), hl3.reshape(2 * NPAD, H),
                         src4, dstp, zeros_h)
    mu_out, ls_out = _k2_final(accm2, accl2, hm3, hl3, dv,
                               bmu[None, :], bls[None, :])
    return mu_out[:N], ls_out[:N]
